# baseline (speedup 1.0000x reference)
import functools

import jax
import jax.numpy as jnp
from jax import lax
from jax.experimental import pallas as pl
from jax.experimental.pallas import tpu as pltpu
from jax.experimental.pallas import tpu_sc as plsc

B0, B1, C, NB = 4096, 512, 4, 16
N_GROUPS = B0 * B1
NC, NS = 2, 16
NW = NC * NS
GROUPS_PER_W = N_GROUPS // NW
G = 2048
CHUNKS = GROUPS_PER_W // G
IN_CHUNK = G * C
OUT_CHUNK = G * NB
STEPS = IN_CHUNK // 16
UNROLL = 8


def _body(x_hbm, z_hbm, out_hbm, iv0, iv1, ov0, ov1, is0, is1, os0, os1):
    c = lax.axis_index("c")
    s = lax.axis_index("s")
    wid = s * NC + c
    base_g = wid * GROUPS_PER_W

    in_bufs = (iv0, iv1)
    out_bufs = (ov0, ov1)
    in_sems = (is0, is1)
    out_sems = (os0, os1)

    z0 = pltpu.async_copy(z_hbm, ov0, os0)
    z1 = pltpu.async_copy(z_hbm, ov1, os1)

    i16 = lax.iota(jnp.int32, 16)
    pattern = i16 + 3 * (i16 - lax.rem(i16, 4)) + 1

    in_copies = [None] * CHUNKS
    out_copies = [None] * CHUNKS
    in_copies[0] = pltpu.async_copy(
        x_hbm.at[pl.ds(base_g * C, IN_CHUNK)], iv0, is0)
    z0.wait()
    z1.wait()

    for t in range(CHUNKS):
        b = t % 2
        if t + 1 < CHUNKS:
            in_copies[t + 1] = pltpu.async_copy(
                x_hbm.at[pl.ds((base_g + (t + 1) * G) * C, IN_CHUNK)],
                in_bufs[(t + 1) % 2], in_sems[(t + 1) % 2])
        in_copies[t].wait()
        if t >= 2:
            out_copies[t - 2].wait()

        iv = in_bufs[b]
        ov = out_bufs[b]

        def step(j, carry, iv=iv, ov=ov):
            base = j * (16 * UNROLL)
            for u in range(UNROLL):
                v = iv[pl.ds(base + u * 16, 16)]
                idx = pattern + (base + u * 16) * 4
                plsc.store_scatter(ov, [idx], v)
            return carry

        lax.fori_loop(0, STEPS // UNROLL, step, 0, unroll=False)

        out_copies[t] = pltpu.async_copy(
            ov, out_hbm.at[pl.ds((base_g + t * G) * NB, OUT_CHUNK)],
            out_sems[b])

    out_copies[CHUNKS - 2].wait()
    out_copies[CHUNKS - 1].wait()


@functools.partial(jax.jit, donate_argnums=())
def _run(x_flat, zeros_chunk):
    mesh = plsc.VectorSubcoreMesh(
        core_axis_name="c", subcore_axis_name="s",
        num_cores=NC, num_subcores=NS)
    f = pl.kernel(
        _body,
        out_type=jax.ShapeDtypeStruct((N_GROUPS * NB,), jnp.float32),
        mesh=mesh,
        scratch_types=[
            pltpu.VMEM((IN_CHUNK,), jnp.float32),
            pltpu.VMEM((IN_CHUNK,), jnp.float32),
            pltpu.VMEM((OUT_CHUNK,), jnp.float32),
            pltpu.VMEM((OUT_CHUNK,), jnp.float32),
            pltpu.SemaphoreType.DMA,
            pltpu.SemaphoreType.DMA,
            pltpu.SemaphoreType.DMA,
            pltpu.SemaphoreType.DMA,
        ],
        compiler_params=pltpu.CompilerParams(needs_layout_passes=False),
    )
    return f(x_flat, zeros_chunk)


def kernel(inputs):
    x_flat = inputs.reshape(-1)
    zeros_chunk = jnp.zeros((OUT_CHUNK,), jnp.float32)
    out = _run(x_flat, zeros_chunk)
    return out.reshape(B0, B1, NB)
